# baseline (speedup 1.0000x reference)
import functools

import jax
import jax.numpy as jnp
from jax import lax
from jax.experimental import pallas as pl
from jax.experimental.pallas import tpu as pltpu
from jax.experimental.pallas import tpu_sc as plsc

NC = 2
NS = 16
NW = NC * NS
LANES = 16
NBUF = 6


def _tree_sum(vals):
    while len(vals) > 1:
        nxt = [a + b for a, b in zip(vals[::2], vals[1::2])]
        if len(vals) % 2:
            nxt.append(vals[-1])
        vals = nxt
    return vals[0]


def _make_kernel(B, K):
    assert B % NW == 0
    b_per_w = B // NW
    C = min(64, b_per_w)
    n_chunks = b_per_w // C
    k_regs = K // LANES
    n_groups = C // LANES

    mesh = plsc.VectorSubcoreMesh(core_axis_name="c", subcore_axis_name="s")

    @functools.partial(
        pl.kernel,
        out_type=jax.ShapeDtypeStruct((B,), jnp.float32),
        mesh=mesh,
        compiler_params=pltpu.CompilerParams(needs_layout_passes=False),
        scratch_types=[
            pltpu.VMEM((b_per_w,), jnp.int32),
            pltpu.VMEM((b_per_w,), jnp.int32),
        ] + [pltpu.VMEM((C, K), jnp.float32) for _ in range(2 * NBUF)] + [
            pltpu.VMEM((b_per_w,), jnp.float32),
            pltpu.VMEM((b_per_w,), jnp.float32),
            pltpu.VMEM((LANES,), jnp.float32),
            pltpu.VMEM((b_per_w,), jnp.float32),
        ] + [pltpu.SemaphoreType.DMA for _ in range(NBUF + 1)],
    )
    def cf_kernel(uidx_hbm, iidx_hbm, uemb_hbm, iemb_hbm, ubias_hbm,
                  ibias_hbm, gbias_hbm, out_hbm,
                  uidx_v, iidx_v, *rest):
        urows = rest[0:NBUF]
        irows = rest[NBUF:2 * NBUF]
        ub_v, ib_v, gb_v, out_v = rest[2 * NBUF:2 * NBUF + 4]
        sems = rest[2 * NBUF + 4:2 * NBUF + 4 + NBUF]
        bias_sem = rest[2 * NBUF + 4 + NBUF]

        wid = lax.axis_index("s") * NC + lax.axis_index("c")
        base = wid * b_per_w
        idx_cps = [
            pltpu.async_copy(uidx_hbm.at[pl.ds(base, b_per_w)], uidx_v,
                             bias_sem),
            pltpu.async_copy(iidx_hbm.at[pl.ds(base, b_per_w)], iidx_v,
                             bias_sem),
            pltpu.async_copy(gbias_hbm, gb_v.at[pl.ds(0, 1)], bias_sem),
        ]
        for cp in idx_cps:
            cp.wait()
        lane = lax.iota(jnp.int32, LANES)

        def issue(c):
            bi = c % NBUF
            u_idx = uidx_v.at[pl.ds(c * C, C)]
            i_idx = iidx_v.at[pl.ds(c * C, C)]
            return [
                pltpu.async_copy(uemb_hbm.at[u_idx], urows[bi], sems[bi]),
                pltpu.async_copy(iemb_hbm.at[i_idx], irows[bi], sems[bi]),
            ]

        pending = [issue(c) for c in range(min(NBUF, n_chunks))]
        bias_cps = []
        for c in range(n_chunks):
            u_idx = uidx_v.at[pl.ds(c * C, C)]
            i_idx = iidx_v.at[pl.ds(c * C, C)]
            bias_cps.append(pltpu.async_copy(
                ubias_hbm.at[u_idx], ub_v.at[pl.ds(c * C, C)], bias_sem))
            bias_cps.append(pltpu.async_copy(
                ibias_hbm.at[i_idx], ib_v.at[pl.ds(c * C, C)], bias_sem))

        for c in range(n_chunks):
            bi = c % NBUF
            for cp in pending[0]:
                cp.wait()
            pending = pending[1:]

            uro, iro = urows[bi], irows[bi]

            @plsc.parallel_loop(0, C, unroll=8)
            def _(e):
                prods = [uro[e, pl.ds(j * LANES, LANES)]
                         * iro[e, pl.ds(j * LANES, LANES)]
                         for j in range(k_regs)]
                tot = plsc.cumsum(_tree_sum(prods))
                plsc.store_scatter(
                    out_v, [jnp.full((LANES,), c * C, jnp.int32) + e], tot,
                    mask=lane == LANES - 1)

            if c + NBUF < n_chunks:
                pending.append(issue(c + NBUF))

        for cp in bias_cps:
            cp.wait()
        gb = gb_v[pl.ds(0, LANES)][0]

        @plsc.parallel_loop(0, b_per_w // LANES)
        def _(g):
            goff = g * LANES
            out_v[pl.ds(goff, LANES)] = (
                out_v[pl.ds(goff, LANES)] + ub_v[pl.ds(goff, LANES)]
                + ib_v[pl.ds(goff, LANES)] + gb)

        pltpu.sync_copy(out_v, out_hbm.at[pl.ds(base, b_per_w)])

    return cf_kernel


def kernel(user_input, item_input, user_emb, item_emb, user_bias, item_bias,
           global_bias):
    B = user_input.shape[0]
    K = user_emb.shape[1]
    k = _make_kernel(B, K)
    return k(user_input.astype(jnp.int32), item_input.astype(jnp.int32),
             user_emb, item_emb,
             user_bias.reshape(-1), item_bias.reshape(-1), global_bias)


# speedup vs baseline: 1.2194x; 1.2194x over previous
import functools

import jax
import jax.numpy as jnp
from jax import lax
from jax.experimental import pallas as pl
from jax.experimental.pallas import tpu as pltpu
from jax.experimental.pallas import tpu_sc as plsc

NC = 2
NS = 16
NW = NC * NS
LANES = 16
NBUF = 6


def _tree_sum(vals):
    while len(vals) > 1:
        nxt = [a + b for a, b in zip(vals[::2], vals[1::2])]
        if len(vals) % 2:
            nxt.append(vals[-1])
        vals = nxt
    return vals[0]


def _make_kernel(B, K):
    assert B % NW == 0
    b_per_w = B // NW
    C = min(64, b_per_w)
    n_chunks = b_per_w // C
    k_regs = K // LANES
    n_groups = C // LANES

    mesh = plsc.VectorSubcoreMesh(core_axis_name="c", subcore_axis_name="s")

    @functools.partial(
        pl.kernel,
        out_type=jax.ShapeDtypeStruct((B,), jnp.float32),
        mesh=mesh,
        compiler_params=pltpu.CompilerParams(needs_layout_passes=False),
        scratch_types=[
            pltpu.VMEM((b_per_w,), jnp.int32),
            pltpu.VMEM((b_per_w,), jnp.int32),
        ] + [pltpu.VMEM((C, K), jnp.float32) for _ in range(2 * NBUF)] + [
            pltpu.VMEM((b_per_w,), jnp.float32),
            pltpu.VMEM((b_per_w,), jnp.float32),
            pltpu.VMEM((LANES,), jnp.float32),
            pltpu.VMEM((b_per_w,), jnp.float32),
        ] + [pltpu.SemaphoreType.DMA for _ in range(NBUF + 1)],
    )
    def cf_kernel(uidx_hbm, iidx_hbm, uemb_hbm, iemb_hbm, ubias_hbm,
                  ibias_hbm, gbias_hbm, out_hbm,
                  uidx_v, iidx_v, *rest):
        urows = rest[0:NBUF]
        irows = rest[NBUF:2 * NBUF]
        ub_v, ib_v, gb_v, out_v = rest[2 * NBUF:2 * NBUF + 4]
        sems = rest[2 * NBUF + 4:2 * NBUF + 4 + NBUF]
        bias_sem = rest[2 * NBUF + 4 + NBUF]

        wid = lax.axis_index("s") * NC + lax.axis_index("c")
        base = wid * b_per_w
        idx_cps = [
            pltpu.async_copy(uidx_hbm.at[pl.ds(base, b_per_w)], uidx_v,
                             bias_sem),
            pltpu.async_copy(iidx_hbm.at[pl.ds(base, b_per_w)], iidx_v,
                             bias_sem),
            pltpu.async_copy(gbias_hbm, gb_v.at[pl.ds(0, 1)], bias_sem),
        ]
        for cp in idx_cps:
            cp.wait()
        lane = lax.iota(jnp.int32, LANES)

        def issue(c):
            bi = c % NBUF
            u_idx = uidx_v.at[pl.ds(c * C, C)]
            i_idx = iidx_v.at[pl.ds(c * C, C)]
            return [
                pltpu.async_copy(uemb_hbm.at[u_idx], urows[bi], sems[bi]),
                pltpu.async_copy(iemb_hbm.at[i_idx], irows[bi], sems[bi]),
            ]

        pending = [issue(c) for c in range(min(NBUF, n_chunks))]
        bias_cps = []
        for c in range(n_chunks):
            u_idx = uidx_v.at[pl.ds(c * C, C)]
            i_idx = iidx_v.at[pl.ds(c * C, C)]
            bias_cps.append(pltpu.async_copy(
                ubias_hbm.at[u_idx], ub_v.at[pl.ds(c * C, C)], bias_sem))
            bias_cps.append(pltpu.async_copy(
                ibias_hbm.at[i_idx], ib_v.at[pl.ds(c * C, C)], bias_sem))

        for c in range(n_chunks):
            bi = c % NBUF
            for cp in pending[0]:
                cp.wait()
            pending = pending[1:]

            uro, iro = urows[bi], irows[bi]

            @plsc.parallel_loop(0, C, unroll=4)
            def _(e):
                prods = [uro[e, pl.ds(j * LANES, LANES)]
                         * iro[e, pl.ds(j * LANES, LANES)]
                         for j in range(k_regs)]
                tot = plsc.cumsum(_tree_sum(prods))
                plsc.store_scatter(
                    out_v, [jnp.full((LANES,), c * C, jnp.int32) + e], tot,
                    mask=lane == LANES - 1)

            if c + NBUF < n_chunks:
                pending.append(issue(c + NBUF))

        for cp in bias_cps:
            cp.wait()
        gb = gb_v[pl.ds(0, LANES)][0]

        @plsc.parallel_loop(0, b_per_w // LANES)
        def _(g):
            goff = g * LANES
            out_v[pl.ds(goff, LANES)] = (
                out_v[pl.ds(goff, LANES)] + ub_v[pl.ds(goff, LANES)]
                + ib_v[pl.ds(goff, LANES)] + gb)

        pltpu.sync_copy(out_v, out_hbm.at[pl.ds(base, b_per_w)])

    return cf_kernel


def kernel(user_input, item_input, user_emb, item_emb, user_bias, item_bias,
           global_bias):
    B = user_input.shape[0]
    K = user_emb.shape[1]
    k = _make_kernel(B, K)
    return k(user_input.astype(jnp.int32), item_input.astype(jnp.int32),
             user_emb, item_emb,
             user_bias.reshape(-1), item_bias.reshape(-1), global_bias)


# speedup vs baseline: 1.2212x; 1.0015x over previous
import functools

import jax
import jax.numpy as jnp
from jax import lax
from jax.experimental import pallas as pl
from jax.experimental.pallas import tpu as pltpu
from jax.experimental.pallas import tpu_sc as plsc

NC = 2
NS = 16
NW = NC * NS
LANES = 16
NBUF = 6


def _tree_sum(vals):
    while len(vals) > 1:
        nxt = [a + b for a, b in zip(vals[::2], vals[1::2])]
        if len(vals) % 2:
            nxt.append(vals[-1])
        vals = nxt
    return vals[0]


def _make_kernel(B, K):
    assert B % NW == 0
    b_per_w = B // NW
    C = min(64, b_per_w)
    n_chunks = b_per_w // C
    k_regs = K // LANES

    mesh = plsc.VectorSubcoreMesh(core_axis_name="c", subcore_axis_name="s")

    @functools.partial(
        pl.kernel,
        out_type=jax.ShapeDtypeStruct((B,), jnp.float32),
        mesh=mesh,
        compiler_params=pltpu.CompilerParams(needs_layout_passes=False),
        scratch_types=[
            pltpu.VMEM((b_per_w,), jnp.int32),
            pltpu.VMEM((b_per_w,), jnp.int32),
        ] + [pltpu.VMEM((C, K), jnp.float32) for _ in range(2 * NBUF)] + [
            pltpu.VMEM((b_per_w,), jnp.float32),
            pltpu.VMEM((b_per_w,), jnp.float32),
            pltpu.VMEM((LANES,), jnp.float32),
            pltpu.VMEM((b_per_w,), jnp.float32),
        ] + [pltpu.SemaphoreType.DMA for _ in range(NBUF + 1)],
    )
    def cf_kernel(uidx_hbm, iidx_hbm, uemb_hbm, iemb_hbm, ubias_hbm,
                  ibias_hbm, gbias_hbm, out_hbm,
                  uidx_v, iidx_v, *rest):
        urows = rest[0:NBUF]
        irows = rest[NBUF:2 * NBUF]
        ub_v, ib_v, gb_v, out_v = rest[2 * NBUF:2 * NBUF + 4]
        sems = rest[2 * NBUF + 4:2 * NBUF + 4 + NBUF]
        bias_sem = rest[2 * NBUF + 4 + NBUF]

        wid = lax.axis_index("s") * NC + lax.axis_index("c")
        base = wid * b_per_w
        idx_cps = [
            pltpu.async_copy(uidx_hbm.at[pl.ds(base, b_per_w)], uidx_v,
                             bias_sem),
            pltpu.async_copy(iidx_hbm.at[pl.ds(base, b_per_w)], iidx_v,
                             bias_sem),
            pltpu.async_copy(gbias_hbm, gb_v.at[pl.ds(0, 1)], bias_sem),
        ]
        for cp in idx_cps:
            cp.wait()
        lane = lax.iota(jnp.int32, LANES)

        def issue(c):
            bi = c % NBUF
            u_idx = uidx_v.at[pl.ds(c * C, C)]
            i_idx = iidx_v.at[pl.ds(c * C, C)]
            return [
                pltpu.async_copy(uemb_hbm.at[u_idx], urows[bi], sems[bi]),
                pltpu.async_copy(iemb_hbm.at[i_idx], irows[bi], sems[bi]),
            ]

        pending = [issue(c) for c in range(min(NBUF, n_chunks))]
        bias_cps = []
        for c in range(n_chunks):
            u_idx = uidx_v.at[pl.ds(c * C, C)]
            i_idx = iidx_v.at[pl.ds(c * C, C)]
            bias_cps.append(pltpu.async_copy(
                ubias_hbm.at[u_idx], ub_v.at[pl.ds(c * C, C)], bias_sem))
            bias_cps.append(pltpu.async_copy(
                ibias_hbm.at[i_idx], ib_v.at[pl.ds(c * C, C)], bias_sem))

        for c in range(n_chunks):
            bi = c % NBUF
            for cp in pending[0]:
                cp.wait()
            pending = pending[1:]

            uro, iro = urows[bi], irows[bi]

            @plsc.parallel_loop(0, C, unroll=4)
            def _(e):
                prods = [uro[e, pl.ds(j * LANES, LANES)]
                         * iro[e, pl.ds(j * LANES, LANES)]
                         for j in range(k_regs)]
                tot = plsc.cumsum(_tree_sum(prods))
                plsc.store_scatter(
                    out_v, [jnp.full((LANES,), c * C, jnp.int32) + e], tot,
                    mask=lane == LANES - 1)

            if c + NBUF < n_chunks:
                pending.append(issue(c + NBUF))

        for cp in bias_cps:
            cp.wait()
        gb = gb_v[pl.ds(0, LANES)][0]

        @plsc.parallel_loop(0, b_per_w // LANES)
        def _(g):
            goff = g * LANES
            out_v[pl.ds(goff, LANES)] = (
                out_v[pl.ds(goff, LANES)] + ub_v[pl.ds(goff, LANES)]
                + ib_v[pl.ds(goff, LANES)] + gb)

        pltpu.sync_copy(out_v, out_hbm.at[pl.ds(base, b_per_w)])

    return cf_kernel


def kernel(user_input, item_input, user_emb, item_emb, user_bias, item_bias,
           global_bias):
    B = user_input.shape[0]
    K = user_emb.shape[1]
    k = _make_kernel(B, K)
    return k(user_input.astype(jnp.int32), item_input.astype(jnp.int32),
             user_emb, item_emb,
             user_bias.reshape(-1), item_bias.reshape(-1), global_bias)
